# Initial kernel scaffold; baseline (speedup 1.0000x reference)
import jax
import jax.numpy as jnp
from jax.experimental import pallas as pl


def kernel(x, edge_index, W1, b1, W2, b2):
    raise NotImplementedError("write your pallas kernel here")


# speedup vs baseline: 15.7488x; 15.7488x over previous
import functools

import jax
import jax.numpy as jnp
from jax import lax
from jax.experimental import pallas as pl
from jax.experimental.pallas import tpu as pltpu
from jax.experimental.pallas import tpu_sc as plsc

N = 10000
E = 320000
OUT = 64
NPAD = 10240
NC = 2
NS = 16
NW = NC * NS
EPT = E // NW
CHUNK = 128
NFULL = EPT // CHUNK
TAIL = EPT - NFULL * CHUNK
RA = 624
RREM_OFF = RA * NS
RREM = N - RREM_OFF
HPT = NPAD // NS
HW = 16

_mesh = lambda: plsc.VectorSubcoreMesh(core_axis_name="c", subcore_axis_name="s")


@functools.partial(
    pl.kernel,
    mesh=_mesh(),
    out_type=jax.ShapeDtypeStruct((NC, NPAD, HW), jnp.float32),
    scratch_types=[
        pltpu.VMEM((CHUNK, HW), jnp.float32),
        pltpu.VMEM((1, CHUNK), jnp.int32),
        pltpu.VMEM((1, TAIL), jnp.int32),
        pltpu.VMEM_SHARED((NPAD, HW), jnp.float32),
    ],
)
def _hist(dst_hbm, ones_hbm, out_hbm, ones_v, didx, didxt, accum):
    c = lax.axis_index("c")
    s = lax.axis_index("s")
    wid = c * NS + s
    pltpu.sync_copy(ones_hbm.at[pl.ds(0, CHUNK)], ones_v)
    pltpu.sync_copy(ones_hbm, accum.at[pl.ds(s * HPT, HPT)])
    plsc.subcore_barrier()
    base = wid * EPT

    def body(j, carry):
        off = base + j * CHUNK
        pltpu.sync_copy(dst_hbm.at[pl.ds(off, CHUNK)], didx.at[0])
        pltpu.sync_copy(ones_v, accum.at[didx.at[0]], add=True)
        return carry

    lax.fori_loop(0, NFULL, body, 0)
    offt = base + NFULL * CHUNK
    pltpu.sync_copy(dst_hbm.at[pl.ds(offt, TAIL)], didxt.at[0])
    pltpu.sync_copy(ones_v.at[pl.ds(0, TAIL)], accum.at[didxt.at[0]], add=True)
    plsc.subcore_barrier()
    pltpu.sync_copy(accum.at[pl.ds(s * HPT, HPT)],
                    out_hbm.at[c, pl.ds(s * HPT, HPT)])


def _make_agg(C):
    @functools.partial(
        pl.kernel,
        mesh=_mesh(),
        out_type=jax.ShapeDtypeStruct((NC, N, C), jnp.float32),
        scratch_types=[
            pltpu.VMEM((CHUNK,), jnp.int32),
            pltpu.VMEM((1, CHUNK), jnp.int32),
            pltpu.VMEM((CHUNK, C), jnp.float32),
            pltpu.VMEM((TAIL,), jnp.int32),
            pltpu.VMEM((1, TAIL), jnp.int32),
            pltpu.VMEM((TAIL, C), jnp.float32),
            pltpu.VMEM_SHARED((N, C), jnp.float32),
            pltpu.SemaphoreType.DMA,
        ],
    )
    def agg(src_hbm, dst_hbm, y_hbm, out_hbm,
            sidx, didx, rows, sidxt, didxt, rowst, accum, sem):
        c = lax.axis_index("c")
        s = lax.axis_index("s")
        wid = c * NS + s
        pltpu.sync_copy(y_hbm.at[pl.ds(s * RA, RA)],
                        accum.at[pl.ds(s * RA, RA)])

        @pl.when(s == NS - 1)
        def _():
            pltpu.sync_copy(y_hbm.at[pl.ds(RREM_OFF, RREM)],
                            accum.at[pl.ds(RREM_OFF, RREM)])

        plsc.subcore_barrier()
        base = wid * EPT

        def body(j, carry):
            off = base + j * CHUNK
            pltpu.sync_copy(src_hbm.at[pl.ds(off, CHUNK)], sidx)
            pltpu.sync_copy(dst_hbm.at[pl.ds(off, CHUNK)], didx.at[0])
            pltpu.async_copy(y_hbm.at[sidx], rows, sem).wait()
            pltpu.sync_copy(rows, accum.at[didx.at[0]], add=True)
            return carry

        lax.fori_loop(0, NFULL, body, 0)
        offt = base + NFULL * CHUNK
        pltpu.sync_copy(src_hbm.at[pl.ds(offt, TAIL)], sidxt)
        pltpu.sync_copy(dst_hbm.at[pl.ds(offt, TAIL)], didxt.at[0])
        pltpu.async_copy(y_hbm.at[sidxt], rowst, sem).wait()
        pltpu.sync_copy(rowst, accum.at[didxt.at[0]], add=True)
        plsc.subcore_barrier()
        pltpu.sync_copy(accum.at[pl.ds(s * RA, RA)],
                        out_hbm.at[c, pl.ds(s * RA, RA)])

        @pl.when(s == NS - 1)
        def _():
            pltpu.sync_copy(accum.at[pl.ds(RREM_OFF, RREM)],
                            out_hbm.at[c, pl.ds(RREM_OFF, RREM)])

    return agg


_agg128 = _make_agg(128)


BR = 1000
GRID = N // BR


def _dinv_blk(d_ref):
    deg = d_ref[0, :, 0:1] + d_ref[1, :, 0:1] - 1.0
    return lax.rsqrt(deg)


def _mm1_body(x_ref, w_ref, d_ref, o_ref):
    y = jnp.dot(x_ref[...], w_ref[...], preferred_element_type=jnp.float32)
    o_ref[...] = y * _dinv_blk(d_ref)


def _mm1(x, W1, degp):
    return pl.pallas_call(
        _mm1_body,
        grid=(GRID,),
        in_specs=[
            pl.BlockSpec((BR, 128), lambda i: (i, 0)),
            pl.BlockSpec((128, 128), lambda i: (0, 0)),
            pl.BlockSpec((NC, BR, HW), lambda i: (0, i, 0)),
        ],
        out_specs=pl.BlockSpec((BR, 128), lambda i: (i, 0)),
        out_shape=jax.ShapeDtypeStruct((N, 128), jnp.float32),
    )(x, W1, degp)


def _mm2_body(p_ref, y1_ref, d_ref, b_ref, w_ref, o_ref):
    dinv = _dinv_blk(d_ref)
    y1 = y1_ref[...]
    agg = p_ref[0] + p_ref[1] - y1
    h = jnp.maximum(agg * dinv + b_ref[...][None, :], 0.0)
    o_ref[...] = jnp.dot(h, w_ref[...], preferred_element_type=jnp.float32) * dinv


def _mm2(p1, y1, degp, b1, W2):
    return pl.pallas_call(
        _mm2_body,
        grid=(GRID,),
        in_specs=[
            pl.BlockSpec((NC, BR, 128), lambda i: (0, i, 0)),
            pl.BlockSpec((BR, 128), lambda i: (i, 0)),
            pl.BlockSpec((NC, BR, HW), lambda i: (0, i, 0)),
            pl.BlockSpec((128,), lambda i: (0,)),
            pl.BlockSpec((128, 128), lambda i: (0, 0)),
        ],
        out_specs=pl.BlockSpec((BR, 128), lambda i: (i, 0)),
        out_shape=jax.ShapeDtypeStruct((N, 128), jnp.float32),
    )(p1, y1, degp, b1, W2)


def _final_body(p_ref, y2_ref, d_ref, b_ref, o_ref):
    dinv = _dinv_blk(d_ref)
    o = (p_ref[0, :, :64] + p_ref[1, :, :64] - y2_ref[:, :64]) * dinv \
        + b_ref[...][None, :]
    m = jnp.max(o, axis=1, keepdims=True)
    z = o - m
    o_ref[...] = z - jnp.log(jnp.sum(jnp.exp(z), axis=1, keepdims=True))


def _final(p2, y2, degp, b2):
    return pl.pallas_call(
        _final_body,
        grid=(GRID,),
        in_specs=[
            pl.BlockSpec((NC, BR, 128), lambda i: (0, i, 0)),
            pl.BlockSpec((BR, 128), lambda i: (i, 0)),
            pl.BlockSpec((NC, BR, HW), lambda i: (0, i, 0)),
            pl.BlockSpec((64,), lambda i: (0,)),
        ],
        out_specs=pl.BlockSpec((BR, 64), lambda i: (i, 0)),
        out_shape=jax.ShapeDtypeStruct((N, 64), jnp.float32),
    )(p2, y2, degp, b2)


def kernel(x, edge_index, W1, b1, W2, b2):
    ei = edge_index.astype(jnp.int32)
    src = ei[0]
    dst = ei[1]
    ones = jnp.ones((HPT, HW), jnp.float32)
    W2p = jnp.pad(W2, ((0, 0), (0, 128 - OUT)))
    degp = _hist(dst, ones)
    y1 = _mm1(x, W1, degp)
    p1 = _agg128(src, dst, y1)
    y2 = _mm2(p1, y1, degp, b1, W2p)
    p2 = _agg128(src, dst, y2)
    return _final(p2, y2, degp, b2)


# speedup vs baseline: 27.6704x; 1.7570x over previous
import functools

import jax
import jax.numpy as jnp
from jax import lax
from jax.experimental import pallas as pl
from jax.experimental.pallas import tpu as pltpu
from jax.experimental.pallas import tpu_sc as plsc

N = 10000
E = 320000
OUT = 64
NPAD = 10240
NC = 2
NS = 16
NW = NC * NS
CHUNK = 64
NCH = E // CHUNK
CBASE = NCH // NW
CEXTRA = NCH - CBASE * NW
CMAX = CBASE + 1
RA = 624
RREM_OFF = RA * NS
RREM = N - RREM_OFF
HPT = NPAD // NS
HW = 16

_mesh = lambda: plsc.VectorSubcoreMesh(core_axis_name="c", subcore_axis_name="s")


def _tile_chunks(c, s):
    w = c * NS + s
    start = w * CBASE + jnp.minimum(w, CEXTRA)
    nc = jnp.where(w < CEXTRA, CBASE + 1, CBASE)
    return start, nc


@functools.partial(
    pl.kernel,
    mesh=_mesh(),
    out_type=jax.ShapeDtypeStruct((NC, NPAD, HW), jnp.float32),
    scratch_types=[
        pltpu.VMEM((CHUNK, HW), jnp.float32),
        pltpu.VMEM((CMAX, CHUNK), jnp.int32),
        pltpu.VMEM_SHARED((NPAD, HW), jnp.float32),
        pltpu.SemaphoreType.DMA,
        pltpu.SemaphoreType.DMA,
    ],
)
def _hist(dst_hbm, ones_hbm, out_hbm, ones_v, didx, accum, dsem, ssem):
    c = lax.axis_index("c")
    s = lax.axis_index("s")
    start, nc = _tile_chunks(c, s)

    def _ld(k, carry):
        pltpu.async_copy(dst_hbm.at[pl.ds((start + k) * CHUNK, CHUNK)],
                         didx.at[k], dsem)
        return carry

    lax.fori_loop(0, nc, _ld, 0)
    pltpu.sync_copy(ones_hbm.at[pl.ds(0, CHUNK)], ones_v)
    pltpu.sync_copy(ones_hbm, accum.at[pl.ds(s * HPT, HPT)])

    def _lw(k, carry):
        pltpu.make_async_copy(dst_hbm.at[pl.ds(0, CHUNK)], didx.at[0],
                              dsem).wait()
        return carry

    lax.fori_loop(0, nc, _lw, 0)
    plsc.subcore_barrier()

    def _sc(j, carry):
        pltpu.async_copy(ones_v, accum.at[didx.at[j]], ssem, add=True)
        return carry

    lax.fori_loop(0, nc, _sc, 0)

    def _sw(j, carry):
        pltpu.make_async_copy(ones_v, accum.at[didx.at[0]], ssem).wait()
        return carry

    lax.fori_loop(0, nc, _sw, 0)
    plsc.subcore_barrier()
    pltpu.sync_copy(accum.at[pl.ds(s * HPT, HPT)],
                    out_hbm.at[c, pl.ds(s * HPT, HPT)])


def _make_agg(C):
    @functools.partial(
        pl.kernel,
        mesh=_mesh(),
        out_type=jax.ShapeDtypeStruct((NC, N, C), jnp.float32),
        scratch_types=[
            pltpu.VMEM((CMAX * CHUNK,), jnp.int32),
            pltpu.VMEM((CMAX, CHUNK), jnp.int32),
            pltpu.VMEM((2, CHUNK, C), jnp.float32),
            pltpu.VMEM_SHARED((N, C), jnp.float32),
            pltpu.SemaphoreType.DMA,
            pltpu.SemaphoreType.DMA,
        ],
    )
    def agg(src_hbm, dst_hbm, y_hbm, out_hbm,
            sidx, didx, rows, accum, dsem, gsem):
        c = lax.axis_index("c")
        s = lax.axis_index("s")
        start, nc = _tile_chunks(c, s)

        pltpu.async_copy(src_hbm.at[pl.ds(start * CHUNK, CBASE * CHUNK)],
                         sidx.at[pl.ds(0, CBASE * CHUNK)], dsem)

        @pl.when(nc == CMAX)
        def _():
            pltpu.async_copy(
                src_hbm.at[pl.ds((start + CBASE) * CHUNK, CHUNK)],
                sidx.at[pl.ds(CBASE * CHUNK, CHUNK)], dsem)

        def _ld(k, carry):
            pltpu.async_copy(dst_hbm.at[pl.ds((start + k) * CHUNK, CHUNK)],
                             didx.at[k], dsem)
            return carry

        lax.fori_loop(0, nc, _ld, 0)

        pltpu.sync_copy(y_hbm.at[pl.ds(s * RA, RA)],
                        accum.at[pl.ds(s * RA, RA)])

        @pl.when(s == NS - 1)
        def _():
            pltpu.sync_copy(y_hbm.at[pl.ds(RREM_OFF, RREM)],
                            accum.at[pl.ds(RREM_OFF, RREM)])

        pltpu.make_async_copy(src_hbm.at[pl.ds(0, CBASE * CHUNK)],
                              sidx.at[pl.ds(0, CBASE * CHUNK)], dsem).wait()

        @pl.when(nc == CMAX)
        def _():
            pltpu.make_async_copy(src_hbm.at[pl.ds(0, CHUNK)],
                                  sidx.at[pl.ds(0, CHUNK)], dsem).wait()

        def _lw(k, carry):
            pltpu.make_async_copy(dst_hbm.at[pl.ds(0, CHUNK)], didx.at[0],
                                  dsem).wait()
            return carry

        lax.fori_loop(0, nc, _lw, 0)
        plsc.subcore_barrier()

        def _gather(j, buf):
            pltpu.async_copy(
                y_hbm.at[sidx.at[pl.ds(j * CHUNK, CHUNK)]],
                rows.at[buf], gsem)

        _gather(0, 0)

        def body(j, carry):
            @pl.when(j + 1 < nc)
            def _():
                _gather(j + 1, (j + 1) % 2)

            pltpu.make_async_copy(
                y_hbm.at[sidx.at[pl.ds(0, CHUNK)]],
                rows.at[j % 2], gsem).wait()
            pltpu.sync_copy(rows.at[j % 2], accum.at[didx.at[j]], add=True)
            return carry

        lax.fori_loop(0, nc, body, 0)
        plsc.subcore_barrier()
        pltpu.sync_copy(accum.at[pl.ds(s * RA, RA)],
                        out_hbm.at[c, pl.ds(s * RA, RA)])

        @pl.when(s == NS - 1)
        def _():
            pltpu.sync_copy(accum.at[pl.ds(RREM_OFF, RREM)],
                            out_hbm.at[c, pl.ds(RREM_OFF, RREM)])

    return agg


_agg128 = _make_agg(128)


BR = 1000
GRID = N // BR


def _dinv_blk(d_ref):
    deg = d_ref[0, :, 0:1] + d_ref[1, :, 0:1] - 1.0
    return lax.rsqrt(deg)


def _mm1_body(x_ref, w_ref, d_ref, o_ref):
    y = jnp.dot(x_ref[...], w_ref[...], preferred_element_type=jnp.float32)
    o_ref[...] = y * _dinv_blk(d_ref)


def _mm1(x, W1, degp):
    return pl.pallas_call(
        _mm1_body,
        grid=(GRID,),
        in_specs=[
            pl.BlockSpec((BR, 128), lambda i: (i, 0)),
            pl.BlockSpec((128, 128), lambda i: (0, 0)),
            pl.BlockSpec((NC, BR, HW), lambda i: (0, i, 0)),
        ],
        out_specs=pl.BlockSpec((BR, 128), lambda i: (i, 0)),
        out_shape=jax.ShapeDtypeStruct((N, 128), jnp.float32),
    )(x, W1, degp)


def _mm2_body(p_ref, y1_ref, d_ref, b_ref, w_ref, o_ref):
    dinv = _dinv_blk(d_ref)
    y1 = y1_ref[...]
    agg = p_ref[0] + p_ref[1] - y1
    h = jnp.maximum(agg * dinv + b_ref[...][None, :], 0.0)
    o_ref[...] = jnp.dot(h, w_ref[...], preferred_element_type=jnp.float32) * dinv


def _mm2(p1, y1, degp, b1, W2):
    return pl.pallas_call(
        _mm2_body,
        grid=(GRID,),
        in_specs=[
            pl.BlockSpec((NC, BR, 128), lambda i: (0, i, 0)),
            pl.BlockSpec((BR, 128), lambda i: (i, 0)),
            pl.BlockSpec((NC, BR, HW), lambda i: (0, i, 0)),
            pl.BlockSpec((128,), lambda i: (0,)),
            pl.BlockSpec((128, 128), lambda i: (0, 0)),
        ],
        out_specs=pl.BlockSpec((BR, 128), lambda i: (i, 0)),
        out_shape=jax.ShapeDtypeStruct((N, 128), jnp.float32),
    )(p1, y1, degp, b1, W2)


def _final_body(p_ref, y2_ref, d_ref, b_ref, o_ref):
    dinv = _dinv_blk(d_ref)
    o = (p_ref[0, :, :64] + p_ref[1, :, :64] - y2_ref[:, :64]) * dinv \
        + b_ref[...][None, :]
    m = jnp.max(o, axis=1, keepdims=True)
    z = o - m
    o_ref[...] = z - jnp.log(jnp.sum(jnp.exp(z), axis=1, keepdims=True))


def _final(p2, y2, degp, b2):
    return pl.pallas_call(
        _final_body,
        grid=(GRID,),
        in_specs=[
            pl.BlockSpec((NC, BR, 128), lambda i: (0, i, 0)),
            pl.BlockSpec((BR, 128), lambda i: (i, 0)),
            pl.BlockSpec((NC, BR, HW), lambda i: (0, i, 0)),
            pl.BlockSpec((64,), lambda i: (0,)),
        ],
        out_specs=pl.BlockSpec((BR, 64), lambda i: (i, 0)),
        out_shape=jax.ShapeDtypeStruct((N, 64), jnp.float32),
    )(p2, y2, degp, b2)


def kernel(x, edge_index, W1, b1, W2, b2):
    ei = edge_index.astype(jnp.int32)
    src = ei[0]
    dst = ei[1]
    ones = jnp.ones((HPT, HW), jnp.float32)
    W2p = jnp.pad(W2, ((0, 0), (0, 128 - OUT)))
    degp = _hist(dst, ones)
    y1 = _mm1(x, W1, degp)
    p1 = _agg128(src, dst, y1)
    y2 = _mm2(p1, y1, degp, b1, W2p)
    p2 = _agg128(src, dst, y2)
    return _final(p2, y2, degp, b2)
